# Initial kernel scaffold; baseline (speedup 1.0000x reference)
import jax
import jax.numpy as jnp
from jax.experimental import pallas as pl


def kernel(x, edge_index, Wl1, Wr1, b1, Wl2, Wr2, b2):
    raise NotImplementedError("write your pallas kernel here")


# speedup vs baseline: 7.7406x; 7.7406x over previous
import functools

import jax
import jax.numpy as jnp
from jax import lax
from jax.experimental import pallas as pl
from jax.experimental.pallas import tpu as pltpu
from jax.experimental.pallas import tpu_sc as plsc

N_NODES = 10000
N_EDGES = 320000
D = 128

NC = 2
NS = 16
NW = NC * NS
C = 128
KB = 4
K = 80
E_PAD = NW * K * C
N_PAD = 10112
ROWS_PER_SUB = N_PAD // NS

_MESH = plsc.VectorSubcoreMesh(core_axis_name="c", subcore_axis_name="s")


def _agg_body(feat, src, dst, zacc, out, src_v, dst_v, rows_a, rows_b,
              acc_sh, gsem_a, gsem_b, ssem_a, ssem_b):
    cid = lax.axis_index("c")
    sid = lax.axis_index("s")
    wid = cid * NS + sid
    r0 = sid * ROWS_PER_SUB

    @pl.when(sid == 0)
    def _():
        pltpu.sync_copy(zacc, acc_sh)
    plsc.subcore_barrier()

    bufs = (rows_a, rows_b)
    gsems = (gsem_a, gsem_b)
    ssems = (ssem_a, ssem_b)

    @pl.loop(0, K, step=KB)
    def _(jb):
        pltpu.sync_copy(src.at[wid, pl.ds(jb, KB)], src_v)
        pltpu.sync_copy(dst.at[wid, pl.ds(jb, KB)], dst_v)

        g = [None, None]
        s = [None, None]
        g[0] = pltpu.async_copy(feat.at[src_v.at[0]], bufs[0], gsems[0])
        for j in range(KB):
            p = j % 2
            g[p].wait()
            if j + 1 < KB:
                q = (j + 1) % 2
                if s[q] is not None:
                    s[q].wait()
                    s[q] = None
                g[q] = pltpu.async_copy(feat.at[src_v.at[j + 1]], bufs[q],
                                        gsems[q])
            s[p] = pltpu.async_copy(bufs[p], acc_sh.at[dst_v.at[j]],
                                    ssems[p], add=True)
        for p in range(2):
            if s[p] is not None:
                s[p].wait()

    plsc.subcore_barrier()
    pltpu.sync_copy(acc_sh.at[pl.ds(r0, ROWS_PER_SUB)],
                    out.at[cid, pl.ds(r0, ROWS_PER_SUB)])


def _sc_aggregate(feat, src, dst):
    scratch = [
        pltpu.VMEM((KB, C), jnp.int32),
        pltpu.VMEM((KB, C), jnp.int32),
        pltpu.VMEM((C, D), jnp.float32),
        pltpu.VMEM((C, D), jnp.float32),
        pltpu.VMEM_SHARED((N_PAD, D), jnp.float32),
        pltpu.SemaphoreType.DMA,
        pltpu.SemaphoreType.DMA,
        pltpu.SemaphoreType.DMA,
        pltpu.SemaphoreType.DMA,
    ]
    kern = pl.kernel(
        _agg_body,
        out_type=jax.ShapeDtypeStruct((NC, N_PAD, D), jnp.float32),
        mesh=_MESH, scratch_types=scratch)
    zacc = jnp.zeros((N_PAD, D), jnp.float32)
    return kern(feat, src, dst, zacc)


def _cnt_body(dst, zacc, ones, cnt_out, dst_v, ones_v, cnt_sh, sem):
    cid = lax.axis_index("c")
    sid = lax.axis_index("s")
    wid = cid * NS + sid
    r0 = sid * ROWS_PER_SUB

    @pl.when(sid == 0)
    def _():
        pltpu.sync_copy(zacc, cnt_sh)
    pltpu.sync_copy(ones, ones_v)
    plsc.subcore_barrier()

    @pl.loop(0, K, step=KB)
    def _(jb):
        pltpu.sync_copy(dst.at[wid, pl.ds(jb, KB)], dst_v)
        for j in range(KB):
            pltpu.sync_copy(ones_v, cnt_sh.at[dst_v.at[j]], add=True)

    plsc.subcore_barrier()
    pltpu.sync_copy(cnt_sh.at[pl.ds(r0, ROWS_PER_SUB)],
                    cnt_out.at[cid, pl.ds(r0, ROWS_PER_SUB)])


def _sc_counts(dst):
    scratch = [
        pltpu.VMEM((KB, C), jnp.int32),
        pltpu.VMEM((C, D), jnp.float32),
        pltpu.VMEM_SHARED((N_PAD, D), jnp.float32),
        pltpu.SemaphoreType.DMA,
    ]
    kern = pl.kernel(
        _cnt_body,
        out_type=jax.ShapeDtypeStruct((NC, N_PAD, D), jnp.float32),
        mesh=_MESH, scratch_types=scratch)
    zacc = jnp.zeros((N_PAD, D), jnp.float32)
    ones = jnp.ones((C, D), jnp.float32)
    return kern(dst, zacc, ones)


def _tc_block(relu, p0_ref, p1_ref, c0_ref, c1_ref, x_ref, wl_ref, wr_ref,
              b_ref, o_ref):
    cnt = c0_ref[:, 0:1] + c1_ref[:, 0:1]
    mean = (p0_ref[...] + p1_ref[...]) / jnp.maximum(cnt, 1.0)
    acc = (jnp.dot(mean, wl_ref[...], preferred_element_type=jnp.float32,
                   precision=lax.Precision.HIGHEST)
           + jnp.dot(x_ref[...], wr_ref[...], preferred_element_type=jnp.float32,
                     precision=lax.Precision.HIGHEST)
           + b_ref[...])
    o_ref[...] = jnp.maximum(acc, 0.0) if relu else acc


def _tc_layer(part, cnt, feat, Wl, Wr, b, relu):
    blk = 1000
    grid = (N_NODES // blk,)
    row_spec = pl.BlockSpec((blk, D), lambda i: (i, 0))
    full = pl.BlockSpec((D, D), lambda i: (0, 0))
    bspec = pl.BlockSpec((1, D), lambda i: (0, 0))
    return pl.pallas_call(
        functools.partial(_tc_block, relu),
        grid=grid,
        in_specs=[row_spec, row_spec, row_spec, row_spec, row_spec, full,
                  full, bspec],
        out_specs=row_spec,
        out_shape=jax.ShapeDtypeStruct((N_NODES, D), jnp.float32),
    )(part[0], part[1], cnt[0], cnt[1], feat, Wl.T, Wr.T, b.reshape(1, D))


def kernel(x, edge_index, Wl1, Wr1, b1, Wl2, Wr2, b2):
    src = edge_index[0].astype(jnp.int32)
    dst = edge_index[1].astype(jnp.int32)
    pad = E_PAD - N_EDGES
    pad_iota = jnp.arange(pad, dtype=jnp.int32)
    src_p = jnp.concatenate([src, pad_iota % N_NODES]).reshape(NW, K, C)
    dst_p = jnp.concatenate(
        [dst, N_NODES + pad_iota % (N_PAD - N_NODES)]).reshape(NW, K, C)

    cnt = _sc_counts(dst_p)
    part1 = _sc_aggregate(x, src_p, dst_p)
    h = _tc_layer(part1, cnt, x, Wl1, Wr1, b1, True)
    part2 = _sc_aggregate(h, src_p, dst_p)
    return _tc_layer(part2, cnt, h, Wl2, Wr2, b2, False)


# speedup vs baseline: 8.5917x; 1.1100x over previous
import functools

import jax
import jax.numpy as jnp
from jax import lax
from jax.experimental import pallas as pl
from jax.experimental.pallas import tpu as pltpu
from jax.experimental.pallas import tpu_sc as plsc

N_NODES = 10000
N_EDGES = 320000
D = 128

NC = 2
NS = 16
NW = NC * NS
C = 128
KB = 16
K = 80
E_PAD = NW * K * C
N_PAD = 10112
ROWS_PER_SUB = N_PAD // NS

_MESH = plsc.VectorSubcoreMesh(core_axis_name="c", subcore_axis_name="s")


def _agg_body(feat, src, dst, zacc, out, src_v, dst_v, rows_a, rows_b,
              acc_sh, gsem_a, gsem_b, ssem_a, ssem_b):
    cid = lax.axis_index("c")
    sid = lax.axis_index("s")
    wid = cid * NS + sid
    r0 = sid * ROWS_PER_SUB

    @pl.when(sid == 0)
    def _():
        pltpu.sync_copy(zacc, acc_sh)
    plsc.subcore_barrier()

    bufs = (rows_a, rows_b)
    gsems = (gsem_a, gsem_b)
    ssems = (ssem_a, ssem_b)

    @pl.loop(0, K, step=KB)
    def _(jb):
        pltpu.sync_copy(src.at[wid, pl.ds(jb, KB)], src_v)
        pltpu.sync_copy(dst.at[wid, pl.ds(jb, KB)], dst_v)

        g = [None, None]
        s = [None, None]
        g[0] = pltpu.async_copy(feat.at[src_v.at[0]], bufs[0], gsems[0])
        for j in range(KB):
            p = j % 2
            g[p].wait()
            if j + 1 < KB:
                q = (j + 1) % 2
                if s[q] is not None:
                    s[q].wait()
                    s[q] = None
                g[q] = pltpu.async_copy(feat.at[src_v.at[j + 1]], bufs[q],
                                        gsems[q])
            s[p] = pltpu.async_copy(bufs[p], acc_sh.at[dst_v.at[j]],
                                    ssems[p], add=True)
        for p in range(2):
            if s[p] is not None:
                s[p].wait()

    plsc.subcore_barrier()
    pltpu.sync_copy(acc_sh.at[pl.ds(r0, ROWS_PER_SUB)],
                    out.at[cid, pl.ds(r0, ROWS_PER_SUB)])


def _sc_aggregate(feat, src, dst):
    scratch = [
        pltpu.VMEM((KB, C), jnp.int32),
        pltpu.VMEM((KB, C), jnp.int32),
        pltpu.VMEM((C, D), jnp.float32),
        pltpu.VMEM((C, D), jnp.float32),
        pltpu.VMEM_SHARED((N_PAD, D), jnp.float32),
        pltpu.SemaphoreType.DMA,
        pltpu.SemaphoreType.DMA,
        pltpu.SemaphoreType.DMA,
        pltpu.SemaphoreType.DMA,
    ]
    kern = pl.kernel(
        _agg_body,
        out_type=jax.ShapeDtypeStruct((NC, N_PAD, D), jnp.float32),
        mesh=_MESH, scratch_types=scratch)
    zacc = jnp.zeros((N_PAD, D), jnp.float32)
    return kern(feat, src, dst, zacc)


def _cnt_body(dst, zacc, ones, cnt_out, dst_v, ones_v, cnt_sh, sem):
    cid = lax.axis_index("c")
    sid = lax.axis_index("s")
    wid = cid * NS + sid
    r0 = sid * ROWS_PER_SUB

    @pl.when(sid == 0)
    def _():
        pltpu.sync_copy(zacc, cnt_sh)
    pltpu.sync_copy(ones, ones_v)
    plsc.subcore_barrier()

    @pl.loop(0, K, step=KB)
    def _(jb):
        pltpu.sync_copy(dst.at[wid, pl.ds(jb, KB)], dst_v)
        descs = [pltpu.async_copy(ones_v, cnt_sh.at[dst_v.at[j]], sem,
                                  add=True) for j in range(KB)]
        for dsc in descs:
            dsc.wait()

    plsc.subcore_barrier()
    pltpu.sync_copy(cnt_sh.at[pl.ds(r0, ROWS_PER_SUB)],
                    cnt_out.at[cid, pl.ds(r0, ROWS_PER_SUB)])


def _sc_counts(dst):
    scratch = [
        pltpu.VMEM((KB, C), jnp.int32),
        pltpu.VMEM((C, D), jnp.float32),
        pltpu.VMEM_SHARED((N_PAD, D), jnp.float32),
        pltpu.SemaphoreType.DMA,
    ]
    kern = pl.kernel(
        _cnt_body,
        out_type=jax.ShapeDtypeStruct((NC, N_PAD, D), jnp.float32),
        mesh=_MESH, scratch_types=scratch)
    zacc = jnp.zeros((N_PAD, D), jnp.float32)
    ones = jnp.ones((C, D), jnp.float32)
    return kern(dst, zacc, ones)


def _tc_block(relu, p0_ref, p1_ref, c0_ref, c1_ref, x_ref, wl_ref, wr_ref,
              b_ref, o_ref):
    cnt = c0_ref[:, 0:1] + c1_ref[:, 0:1]
    mean = (p0_ref[...] + p1_ref[...]) / jnp.maximum(cnt, 1.0)
    acc = (jnp.dot(mean, wl_ref[...], preferred_element_type=jnp.float32,
                   precision=lax.Precision.HIGHEST)
           + jnp.dot(x_ref[...], wr_ref[...], preferred_element_type=jnp.float32,
                     precision=lax.Precision.HIGHEST)
           + b_ref[...])
    o_ref[...] = jnp.maximum(acc, 0.0) if relu else acc


def _tc_layer(part, cnt, feat, Wl, Wr, b, relu):
    blk = 1000
    grid = (N_NODES // blk,)
    row_spec = pl.BlockSpec((blk, D), lambda i: (i, 0))
    full = pl.BlockSpec((D, D), lambda i: (0, 0))
    bspec = pl.BlockSpec((1, D), lambda i: (0, 0))
    return pl.pallas_call(
        functools.partial(_tc_block, relu),
        grid=grid,
        in_specs=[row_spec, row_spec, row_spec, row_spec, row_spec, full,
                  full, bspec],
        out_specs=row_spec,
        out_shape=jax.ShapeDtypeStruct((N_NODES, D), jnp.float32),
    )(part[0], part[1], cnt[0], cnt[1], feat, Wl.T, Wr.T, b.reshape(1, D))


def kernel(x, edge_index, Wl1, Wr1, b1, Wl2, Wr2, b2):
    src = edge_index[0].astype(jnp.int32)
    dst = edge_index[1].astype(jnp.int32)
    pad = E_PAD - N_EDGES
    pad_iota = jnp.arange(pad, dtype=jnp.int32)
    src_p = jnp.concatenate([src, pad_iota % N_NODES]).reshape(NW, K, C)
    dst_p = jnp.concatenate(
        [dst, N_NODES + pad_iota % (N_PAD - N_NODES)]).reshape(NW, K, C)

    cnt = _sc_counts(dst_p)
    part1 = _sc_aggregate(x, src_p, dst_p)
    h = _tc_layer(part1, cnt, x, Wl1, Wr1, b1, True)
    part2 = _sc_aggregate(h, src_p, dst_p)
    return _tc_layer(part2, cnt, h, Wl2, Wr2, b2, False)


# speedup vs baseline: 8.9742x; 1.0445x over previous
import functools

import jax
import jax.numpy as jnp
from jax import lax
from jax.experimental import pallas as pl
from jax.experimental.pallas import tpu as pltpu
from jax.experimental.pallas import tpu_sc as plsc

N_NODES = 10000
N_EDGES = 320000
D = 128

NC = 2
NS = 16
NW = NC * NS
C = 128
KB = 16
K = 80
E_PAD = NW * K * C
N_PAD = 10112
ROWS_PER_SUB = N_PAD // NS

_MESH = plsc.VectorSubcoreMesh(core_axis_name="c", subcore_axis_name="s")


def _agg_body(feat, src, dst, zacc, out, src_v, dst_v, rows_a, rows_b,
              acc_sh, gsem_a, gsem_b, ssem_a, ssem_b):
    cid = lax.axis_index("c")
    sid = lax.axis_index("s")
    wid = cid * NS + sid
    r0 = sid * ROWS_PER_SUB

    @pl.when(sid == 0)
    def _():
        pltpu.sync_copy(zacc, acc_sh)
    plsc.subcore_barrier()

    bufs = (rows_a, rows_b)
    gsems = (gsem_a, gsem_b)
    ssems = (ssem_a, ssem_b)

    @pl.loop(0, K, step=KB)
    def _(jb):
        pltpu.sync_copy(src.at[wid, pl.ds(jb, KB)], src_v)
        pltpu.sync_copy(dst.at[wid, pl.ds(jb, KB)], dst_v)

        g = [None, None]
        s = [None, None]
        g[0] = pltpu.async_copy(feat.at[src_v.at[0]], bufs[0], gsems[0])
        for j in range(KB):
            p = j % 2
            g[p].wait()
            if j + 1 < KB:
                q = (j + 1) % 2
                if s[q] is not None:
                    s[q].wait()
                    s[q] = None
                g[q] = pltpu.async_copy(feat.at[src_v.at[j + 1]], bufs[q],
                                        gsems[q])
            s[p] = pltpu.async_copy(bufs[p], acc_sh.at[dst_v.at[j]],
                                    ssems[p], add=True)
        for p in range(2):
            if s[p] is not None:
                s[p].wait()

    plsc.subcore_barrier()
    pltpu.sync_copy(acc_sh.at[pl.ds(r0, ROWS_PER_SUB)],
                    out.at[cid, pl.ds(r0, ROWS_PER_SUB)])


def _sc_aggregate(feat, src, dst):
    scratch = [
        pltpu.VMEM((KB, C), jnp.int32),
        pltpu.VMEM((KB, C), jnp.int32),
        pltpu.VMEM((C, D), jnp.float32),
        pltpu.VMEM((C, D), jnp.float32),
        pltpu.VMEM_SHARED((N_PAD, D), jnp.float32),
        pltpu.SemaphoreType.DMA,
        pltpu.SemaphoreType.DMA,
        pltpu.SemaphoreType.DMA,
        pltpu.SemaphoreType.DMA,
    ]
    kern = pl.kernel(
        _agg_body,
        out_type=jax.ShapeDtypeStruct((NC, N_PAD, D), jnp.float32),
        mesh=_MESH, scratch_types=scratch)
    zacc = jnp.zeros((N_PAD, D), jnp.float32)
    return kern(feat, src, dst, zacc)


def _cnt_body(dst, zacc, ones, cnt_out, dst_v, ones_v, cnt_sh, sem):
    cid = lax.axis_index("c")
    sid = lax.axis_index("s")
    wid = cid * NS + sid
    r0 = sid * ROWS_PER_SUB

    @pl.when(sid == 0)
    def _():
        pltpu.sync_copy(zacc, cnt_sh)
    pltpu.sync_copy(ones, ones_v)
    plsc.subcore_barrier()

    @pl.loop(0, K, step=KB)
    def _(jb):
        pltpu.sync_copy(dst.at[wid, pl.ds(jb, KB)], dst_v)
        descs = [pltpu.async_copy(ones_v, cnt_sh.at[dst_v.at[j]], sem,
                                  add=True) for j in range(KB)]
        for dsc in descs:
            dsc.wait()

    plsc.subcore_barrier()
    pltpu.sync_copy(cnt_sh.at[pl.ds(r0, ROWS_PER_SUB)],
                    cnt_out.at[cid, pl.ds(r0, ROWS_PER_SUB)])


def _sc_counts(dst):
    scratch = [
        pltpu.VMEM((KB, C), jnp.int32),
        pltpu.VMEM((C, D), jnp.float32),
        pltpu.VMEM_SHARED((N_PAD, D), jnp.float32),
        pltpu.SemaphoreType.DMA,
    ]
    kern = pl.kernel(
        _cnt_body,
        out_type=jax.ShapeDtypeStruct((NC, N_PAD, D), jnp.float32),
        mesh=_MESH, scratch_types=scratch)
    zacc = jnp.zeros((N_PAD, D), jnp.float32)
    ones = jnp.ones((C, D), jnp.float32)
    return kern(dst, zacc, ones)


def _tc_block(relu, p0_ref, p1_ref, c0_ref, c1_ref, x_ref, wl_ref, wr_ref,
              b_ref, o_ref):
    cnt = c0_ref[0, :, 0:1] + c1_ref[0, :, 0:1]
    mean = (p0_ref[0] + p1_ref[0]) / jnp.maximum(cnt, 1.0)
    acc = (jnp.dot(mean, wl_ref[...], preferred_element_type=jnp.float32,
                   precision=lax.Precision.HIGHEST)
           + jnp.dot(x_ref[...], wr_ref[...], preferred_element_type=jnp.float32,
                     precision=lax.Precision.HIGHEST)
           + b_ref[...])
    o_ref[...] = jnp.maximum(acc, 0.0) if relu else acc


def _tc_layer(part, cnt, feat, Wl, Wr, b, relu):
    blk = 1000
    grid = (N_NODES // blk,)
    p0_spec = pl.BlockSpec((1, blk, D), lambda i: (0, i, 0))
    p1_spec = pl.BlockSpec((1, blk, D), lambda i: (1, i, 0))
    row_spec = pl.BlockSpec((blk, D), lambda i: (i, 0))
    full = pl.BlockSpec((D, D), lambda i: (0, 0))
    bspec = pl.BlockSpec((1, D), lambda i: (0, 0))
    return pl.pallas_call(
        functools.partial(_tc_block, relu),
        grid=grid,
        in_specs=[p0_spec, p1_spec, p0_spec, p1_spec, row_spec, full,
                  full, bspec],
        out_specs=row_spec,
        out_shape=jax.ShapeDtypeStruct((N_NODES, D), jnp.float32),
    )(part, part, cnt, cnt, feat, Wl.T, Wr.T, b.reshape(1, D))


def kernel(x, edge_index, Wl1, Wr1, b1, Wl2, Wr2, b2):
    src = edge_index[0].astype(jnp.int32)
    dst = edge_index[1].astype(jnp.int32)
    pad = E_PAD - N_EDGES
    pad_iota = jnp.arange(pad, dtype=jnp.int32)
    src_p = jnp.concatenate([src, pad_iota % N_NODES]).reshape(NW, K, C)
    dst_p = jnp.concatenate(
        [dst, N_NODES + pad_iota % (N_PAD - N_NODES)]).reshape(NW, K, C)

    cnt = _sc_counts(dst_p)
    part1 = _sc_aggregate(x, src_p, dst_p)
    h = _tc_layer(part1, cnt, x, Wl1, Wr1, b1, True)
    part2 = _sc_aggregate(h, src_p, dst_p)
    return _tc_layer(part2, cnt, h, Wl2, Wr2, b2, False)


# speedup vs baseline: 9.8357x; 1.0960x over previous
import functools

import jax
import jax.numpy as jnp
from jax import lax
from jax.experimental import pallas as pl
from jax.experimental.pallas import tpu as pltpu
from jax.experimental.pallas import tpu_sc as plsc

N_NODES = 10000
N_EDGES = 320000
D = 128

NC = 2
NS = 16
NW = NC * NS
C = 64
KB = 16
K = 160
NBUF = 4
E_PAD = NW * K * C
N_PAD = 10112
ROWS_PER_SUB = N_PAD // NS

_MESH = plsc.VectorSubcoreMesh(core_axis_name="c", subcore_axis_name="s")


def _agg_body(feat, src, dst, zacc, out, src_v, dst_v, rows_bufs,
              acc_sh, gsems, ssems):
    cid = lax.axis_index("c")
    sid = lax.axis_index("s")
    wid = cid * NS + sid
    r0 = sid * ROWS_PER_SUB

    @pl.when(sid == 0)
    def _():
        pltpu.sync_copy(zacc, acc_sh)
    plsc.subcore_barrier()

    bufs = rows_bufs

    @pl.loop(0, K, step=KB)
    def _(jb):
        pltpu.sync_copy(src.at[wid, pl.ds(jb, KB)], src_v)
        pltpu.sync_copy(dst.at[wid, pl.ds(jb, KB)], dst_v)

        g = [None] * NBUF
        s = [None] * NBUF
        for j in range(min(NBUF - 1, KB)):
            g[j] = pltpu.async_copy(feat.at[src_v.at[j]], bufs[j], gsems[j])
        for j in range(KB):
            p = j % NBUF
            g[p].wait()
            jn = j + NBUF - 1
            if jn < KB:
                q = jn % NBUF
                if s[q] is not None:
                    s[q].wait()
                    s[q] = None
                g[q] = pltpu.async_copy(feat.at[src_v.at[jn]], bufs[q],
                                        gsems[q])
            s[p] = pltpu.async_copy(bufs[p], acc_sh.at[dst_v.at[j]],
                                    ssems[p], add=True)
        for p in range(NBUF):
            if s[p] is not None:
                s[p].wait()

    plsc.subcore_barrier()
    pltpu.sync_copy(acc_sh.at[pl.ds(r0, ROWS_PER_SUB)],
                    out.at[cid, pl.ds(r0, ROWS_PER_SUB)])


def _sc_aggregate(feat, src, dst):
    scratch = [
        pltpu.VMEM((KB, C), jnp.int32),
        pltpu.VMEM((KB, C), jnp.int32),
        [pltpu.VMEM((C, D), jnp.float32) for _ in range(NBUF)],
        pltpu.VMEM_SHARED((N_PAD, D), jnp.float32),
        [pltpu.SemaphoreType.DMA for _ in range(NBUF)],
        [pltpu.SemaphoreType.DMA for _ in range(NBUF)],
    ]
    kern = pl.kernel(
        _agg_body,
        out_type=jax.ShapeDtypeStruct((NC, N_PAD, D), jnp.float32),
        mesh=_MESH, scratch_types=scratch)
    zacc = jnp.zeros((N_PAD, D), jnp.float32)
    return kern(feat, src, dst, zacc)


def _cnt_body(dst, zacc, ones, cnt_out, dst_v, ones_v, cnt_sh, sem):
    cid = lax.axis_index("c")
    sid = lax.axis_index("s")
    wid = cid * NS + sid
    r0 = sid * ROWS_PER_SUB

    @pl.when(sid == 0)
    def _():
        pltpu.sync_copy(zacc, cnt_sh)
    pltpu.sync_copy(ones, ones_v)
    plsc.subcore_barrier()

    @pl.loop(0, K, step=KB)
    def _(jb):
        pltpu.sync_copy(dst.at[wid, pl.ds(jb, KB)], dst_v)
        descs = [pltpu.async_copy(ones_v, cnt_sh.at[dst_v.at[j]], sem,
                                  add=True) for j in range(KB)]
        for dsc in descs:
            dsc.wait()

    plsc.subcore_barrier()
    pltpu.sync_copy(cnt_sh.at[pl.ds(r0, ROWS_PER_SUB)],
                    cnt_out.at[cid, pl.ds(r0, ROWS_PER_SUB)])


def _sc_counts(dst):
    scratch = [
        pltpu.VMEM((KB, C), jnp.int32),
        pltpu.VMEM((C, D), jnp.float32),
        pltpu.VMEM_SHARED((N_PAD, D), jnp.float32),
        pltpu.SemaphoreType.DMA,
    ]
    kern = pl.kernel(
        _cnt_body,
        out_type=jax.ShapeDtypeStruct((NC, N_PAD, D), jnp.float32),
        mesh=_MESH, scratch_types=scratch)
    zacc = jnp.zeros((N_PAD, D), jnp.float32)
    ones = jnp.ones((C, D), jnp.float32)
    return kern(dst, zacc, ones)


def _tc_block(relu, p0_ref, p1_ref, c0_ref, c1_ref, x_ref, wl_ref, wr_ref,
              b_ref, o_ref):
    cnt = c0_ref[0, :, 0:1] + c1_ref[0, :, 0:1]
    mean = (p0_ref[0] + p1_ref[0]) / jnp.maximum(cnt, 1.0)
    acc = (jnp.dot(mean, wl_ref[...], preferred_element_type=jnp.float32,
                   precision=lax.Precision.HIGHEST)
           + jnp.dot(x_ref[...], wr_ref[...], preferred_element_type=jnp.float32,
                     precision=lax.Precision.HIGHEST)
           + b_ref[...])
    o_ref[...] = jnp.maximum(acc, 0.0) if relu else acc


def _tc_layer(part, cnt, feat, Wl, Wr, b, relu):
    blk = 1000
    grid = (N_NODES // blk,)
    p0_spec = pl.BlockSpec((1, blk, D), lambda i: (0, i, 0))
    p1_spec = pl.BlockSpec((1, blk, D), lambda i: (1, i, 0))
    row_spec = pl.BlockSpec((blk, D), lambda i: (i, 0))
    full = pl.BlockSpec((D, D), lambda i: (0, 0))
    bspec = pl.BlockSpec((1, D), lambda i: (0, 0))
    return pl.pallas_call(
        functools.partial(_tc_block, relu),
        grid=grid,
        in_specs=[p0_spec, p1_spec, p0_spec, p1_spec, row_spec, full,
                  full, bspec],
        out_specs=row_spec,
        out_shape=jax.ShapeDtypeStruct((N_NODES, D), jnp.float32),
    )(part, part, cnt, cnt, feat, Wl.T, Wr.T, b.reshape(1, D))


def kernel(x, edge_index, Wl1, Wr1, b1, Wl2, Wr2, b2):
    src = edge_index[0].astype(jnp.int32)
    dst = edge_index[1].astype(jnp.int32)
    pad = E_PAD - N_EDGES
    pad_iota = jnp.arange(pad, dtype=jnp.int32)
    src_p = jnp.concatenate([src, pad_iota % N_NODES]).reshape(NW, K, C)
    dst_p = jnp.concatenate(
        [dst, N_NODES + pad_iota % (N_PAD - N_NODES)]).reshape(NW, K, C)

    cnt = _sc_counts(dst_p)
    part1 = _sc_aggregate(x, src_p, dst_p)
    h = _tc_layer(part1, cnt, x, Wl1, Wr1, b1, True)
    part2 = _sc_aggregate(h, src_p, dst_p)
    return _tc_layer(part2, cnt, h, Wl2, Wr2, b2, False)
